# Initial kernel scaffold; baseline (speedup 1.0000x reference)
import functools

import jax
import jax.numpy as jnp
from jax import lax
from jax.experimental import pallas as pl
from jax.experimental.pallas import tpu as pltpu
from jax.experimental.pallas import tpu_sc as plsc

NEG_INF = -1000000000.0

NC = 2
NS = 16
NW = NC * NS

SUB_S = 4
GROUP_S = 80


def _sc_embed_body(S, Q, n_sent_w, n_qsent_w,
                   tok, qtok, t0, t1, t2, t3, bw, pvec_h, qvec_h, rd_h,
                   e0, e1, e2, e3, u0,
                   i0, i1, rA0, rA1, rA2, rA3, rB0, rB1, rB2, rB3,
                   o0, o1, o2, o3, pp, qq, rdm,
                   si0, si1, sA0, sA1, sA2, sA3, sB0, sB1, sB2, sB3):
    wid = lax.axis_index("s") * NC + lax.axis_index("c")

    pltpu.sync_copy(pvec_h, pp)
    pltpu.sync_copy(qvec_h, qq)
    pltpu.sync_copy(rd_h, rdm)
    rd = [rdm[pl.ds(c * 16, 16)] for c in range(4)]

    tabs = [t0, t1, t2, t3]
    outs = [e0, e1, e2, e3]
    ovs = [o0, o1, o2, o3]
    rbufs = [(rA0, rB0), (rA1, rB1), (rA2, rB2), (rA3, rB3)]
    gsems = [(sA0, sB0), (sA1, sB1), (sA2, sB2), (sA3, sB3)]

    zero = jnp.zeros((16,), jnp.float32)

    def compute_sub(buf, ov, row0):
        init = (tuple(zero for _ in range(SUB_S * 4)),
                tuple(zero for _ in range(SUB_S * 4)))

        def s_body(s, carry):
            accp, accq = list(carry[0]), list(carry[1])
            ps = pp[s, :]
            qs = qq[s, :]
            for j in range(SUB_S):
                for c in range(4):
                    row = buf[j * S + s, pl.ds(c * 16, 16)]
                    a = j * 4 + c
                    accp[a] = accp[a] + row * ps
                    accq[a] = accq[a] + row * qs
            return (tuple(accp), tuple(accq))

        accp, accq = lax.fori_loop(0, S, s_body, init)
        for j in range(SUB_S):
            for c in range(4):
                a = j * 4 + c
                ov[row0 + j, pl.ds(c * 16, 16)] = accp[a] + rd[c] * accq[a]

    n_groups = n_sent_w // GROUP_S
    n_sub = GROUP_S // SUB_S
    base_sent = wid * n_sent_w
    ibufs = (i0, i1)
    isems = (si0, si1)

    pltpu.async_copy(tok.at[pl.ds(base_sent * S, GROUP_S * S)], i0, si0)

    @pl.loop(0, n_groups, step=2)
    def group_loop(g0):
        for half in range(2):
            g = g0 + half
            ib, isem = ibufs[half], isems[half]
            inext, isem_next = ibufs[1 - half], isems[1 - half]
            goff = base_sent + g * GROUP_S
            pltpu.make_async_copy(
                tok.at[pl.ds(goff * S, GROUP_S * S)], ib, isem).wait()

            @pl.when(g + 1 < n_groups)
            def _():
                pltpu.async_copy(
                    tok.at[pl.ds((goff + GROUP_S) * S, GROUP_S * S)],
                    inext, isem_next)

            for t in range(4):
                pltpu.async_copy(tabs[t].at[ib.at[pl.ds(0, SUB_S * S)]],
                                 rbufs[t][0], gsems[t][0])

            @pl.loop(0, n_sub, step=2)
            def sub_loop(k0):
                for slot in range(2):
                    kk = k0 + slot
                    for t in range(4):
                        pltpu.make_async_copy(
                            tabs[t].at[ib.at[pl.ds(kk * SUB_S * S, SUB_S * S)]],
                            rbufs[t][slot], gsems[t][slot]).wait()

                    @pl.when(kk + 1 < n_sub)
                    def _():
                        for t in range(4):
                            pltpu.async_copy(
                                tabs[t].at[
                                    ib.at[pl.ds((kk + 1) * SUB_S * S, SUB_S * S)]],
                                rbufs[t][1 - slot], gsems[t][1 - slot])

                    for t in range(4):
                        compute_sub(rbufs[t][slot], ovs[t], kk * SUB_S)

            for t in range(4):
                pltpu.sync_copy(ovs[t], outs[t].at[pl.ds(goff, GROUP_S)])

    n_subq = n_qsent_w // SUB_S
    qbase = wid * n_qsent_w
    pltpu.sync_copy(qtok.at[pl.ds(qbase * Q, n_qsent_w * Q)], i0)
    pltpu.async_copy(bw.at[i0.at[pl.ds(0, SUB_S * Q)]], rbufs[0][0],
                     gsems[0][0])

    @pl.loop(0, n_subq, step=2)
    def qsub_loop(k0):
        for slot in range(2):
            kk = k0 + slot
            pltpu.make_async_copy(
                bw.at[i0.at[pl.ds(kk * SUB_S * Q, SUB_S * Q)]],
                rbufs[0][slot], gsems[0][slot]).wait()

            @pl.when(kk + 1 < n_subq)
            def _():
                pltpu.async_copy(
                    bw.at[i0.at[pl.ds((kk + 1) * SUB_S * Q, SUB_S * Q)]],
                    rbufs[0][1 - slot], gsems[0][1 - slot])

            compute_sub(rbufs[0][slot], ovs[0], kk * SUB_S)

    pltpu.sync_copy(ovs[0].at[pl.ds(0, n_qsent_w)],
                    u0.at[pl.ds(qbase, n_qsent_w)])


def _sc_embed(tok, qtok, t0, t1, t2, t3, bw, pvec, qvec, rdv,
              n_sent, n_qsent, S, Q, D):
    n_sent_w = n_sent // NW
    n_qsent_w = n_qsent // NW
    mesh = plsc.VectorSubcoreMesh(core_axis_name="c", subcore_axis_name="s")
    out_type = ([jax.ShapeDtypeStruct((n_sent, D), jnp.float32)] * 4
                + [jax.ShapeDtypeStruct((n_qsent, D), jnp.float32)])
    scratch = (
        [pltpu.VMEM((GROUP_S * S,), jnp.int32)] * 2
        + [pltpu.VMEM((SUB_S * S, D), jnp.float32)] * 8
        + [pltpu.VMEM((GROUP_S, D), jnp.float32)] * 4
        + [pltpu.VMEM((S, 16), jnp.float32)] * 2
        + [pltpu.VMEM((D,), jnp.float32)]
        + [pltpu.SemaphoreType.DMA] * 10
    )
    body = functools.partial(_sc_embed_body, S, Q, n_sent_w, n_qsent_w)
    fn = pl.kernel(body, out_type=out_type, mesh=mesh, scratch_types=scratch)
    return fn(tok, qtok, t0, t1, t2, t3, bw, pvec, qvec, rdv)


def _hops_body(e0, e1, e2, e3, story_ref, u_ref, ta_ref, tc_ref, out_ref):
    u = u_ref[...]
    nonpad = jnp.any(story_ref[...] != 0, axis=2)
    es = [e0[...], e1[...], e2[...], e3[...]]
    ta = ta_ref[...]
    tc = tc_ref[...]
    for k in range(3):
        m = es[k] + ta[None]
        c = es[k + 1] + tc[None]
        scores = jnp.sum(m * u[:, None, :], axis=2)
        scores = jnp.where(nonpad, scores, NEG_INF)
        p = jax.nn.softmax(scores, axis=1)
        o = jnp.sum(p[:, :, None] * c, axis=1)
        u = u + o
    out_ref[...] = u


def _hops(e0, e1, e2, e3, story, u0, ta, tc, interpret=False):
    B, M, D = e0.shape
    S = story.shape[2]
    BB = 256
    grid = B // BB
    espec = pl.BlockSpec((BB, M, D), lambda i: (i, 0, 0))
    return pl.pallas_call(
        _hops_body,
        grid=(grid,),
        in_specs=[espec, espec, espec, espec,
                  pl.BlockSpec((BB, M, S), lambda i: (i, 0, 0)),
                  pl.BlockSpec((BB, D), lambda i: (i, 0)),
                  pl.BlockSpec((M, D), lambda i: (0, 0)),
                  pl.BlockSpec((M, D), lambda i: (0, 0))],
        out_specs=pl.BlockSpec((BB, D), lambda i: (i, 0)),
        out_shape=jax.ShapeDtypeStruct((B, D), jnp.float32),
        interpret=interpret,
    )(e0, e1, e2, e3, story, u0, ta, tc)


def _mm_body(u_ref, w_ref, b_ref, out_ref):
    out_ref[...] = lax.dot_general(
        u_ref[...], w_ref[...], (((1,), (1,)), ((), ())),
        preferred_element_type=jnp.float32) + b_ref[...]


def _classifier(u, W, b2d, interpret=False):
    B, D = u.shape
    V = W.shape[0]
    VB = 2048
    grid = pl.cdiv(V, VB)
    return pl.pallas_call(
        _mm_body,
        grid=(grid,),
        in_specs=[pl.BlockSpec((B, D), lambda v: (0, 0)),
                  pl.BlockSpec((VB, D), lambda v: (v, 0)),
                  pl.BlockSpec((1, VB), lambda v: (0, v))],
        out_specs=pl.BlockSpec((B, VB), lambda v: (0, v)),
        out_shape=jax.ShapeDtypeStruct((B, V), jnp.float32),
        interpret=interpret,
    )(u, W, b2d)


def kernel(story, question, A0, C0, C1, C2, Bw, TA, TC, W, b):
    B, M, S = story.shape
    Q = question.shape[1]
    V, D = A0.shape
    assert Q == S, "shared position-encoding factors assume Q == S"

    j = jnp.arange(1, S + 1, dtype=jnp.float32)
    p = 1.0 - j / S
    q = 1.0 - 2.0 * j / S
    kk = jnp.arange(1, D + 1, dtype=jnp.float32)
    r = -(kk / D)
    pvec = jnp.broadcast_to(p[:, None], (S, 16))
    qvec = jnp.broadcast_to(q[:, None], (S, 16))

    tok = story.reshape(B * M * S)
    qtok = question.reshape(B * Q)

    e0, e1, e2, e3, u0 = _sc_embed(tok, qtok, A0, C0, C1, C2, Bw,
                                   pvec, qvec, r, B * M, B, S, Q, D)

    u = _hops(e0.reshape(B, M, D), e1.reshape(B, M, D),
              e2.reshape(B, M, D), e3.reshape(B, M, D),
              story, u0, TA, TC)

    return _classifier(u, W, b.reshape(1, V))


# speedup vs baseline: 11.0042x; 11.0042x over previous
import functools

import jax
import jax.numpy as jnp
from jax import lax
from jax.experimental import pallas as pl
from jax.experimental.pallas import tpu as pltpu
from jax.experimental.pallas import tpu_sc as plsc

NEG_INF = -1000000000.0

NC = 2
NS = 16
NW = NC * NS

SUB_S = 4
GROUP_S = 80


def _sc_embed_body(S, Q, n_sent_w, n_qsent_w,
                   tok, qtok, t0, t1, t2, t3, bw, pvec_h, qvec_h, rd_h,
                   e0, e1, e2, e3, u0,
                   i0, i1, rA0, rA1, rA2, rA3, rB0, rB1, rB2, rB3,
                   o0, o1, o2, o3, pp, qq, rdm,
                   si0, si1, sA0, sA1, sA2, sA3, sB0, sB1, sB2, sB3):
    wid = lax.axis_index("s") * NC + lax.axis_index("c")

    pltpu.sync_copy(pvec_h, pp)
    pltpu.sync_copy(qvec_h, qq)
    pltpu.sync_copy(rd_h, rdm)
    rd = [rdm[pl.ds(c * 16, 16)] for c in range(4)]

    tabs = [t0, t1, t2, t3]
    outs = [e0, e1, e2, e3]
    ovs = [o0, o1, o2, o3]
    rbufs = [(rA0, rB0), (rA1, rB1), (rA2, rB2), (rA3, rB3)]
    gsems = [(sA0, sB0), (sA1, sB1), (sA2, sB2), (sA3, sB3)]

    zero = jnp.zeros((16,), jnp.float32)

    def compute_sub(buf, ov, row0):
        init = (tuple(zero for _ in range(SUB_S * 4)),
                tuple(zero for _ in range(SUB_S * 4)))

        def s_body(s, carry):
            accp, accq = list(carry[0]), list(carry[1])
            ps = pp[s, :]
            qs = qq[s, :]
            for j in range(SUB_S):
                for c in range(4):
                    row = buf[j * S + s, pl.ds(c * 16, 16)]
                    a = j * 4 + c
                    accp[a] = accp[a] + row * ps
                    accq[a] = accq[a] + row * qs
            return (tuple(accp), tuple(accq))

        accp, accq = lax.fori_loop(0, S, s_body, init)
        for j in range(SUB_S):
            for c in range(4):
                a = j * 4 + c
                ov[row0 + j, pl.ds(c * 16, 16)] = accp[a] + rd[c] * accq[a]

    n_groups = n_sent_w // GROUP_S
    n_sub = GROUP_S // SUB_S
    base_sent = wid * n_sent_w
    ibufs = (i0, i1)
    isems = (si0, si1)

    pltpu.async_copy(tok.at[pl.ds(base_sent * S, GROUP_S * S)], i0, si0)

    @pl.loop(0, n_groups, step=2)
    def group_loop(g0):
        for half in range(2):
            g = g0 + half
            ib, isem = ibufs[half], isems[half]
            inext, isem_next = ibufs[1 - half], isems[1 - half]
            goff = base_sent + g * GROUP_S
            pltpu.make_async_copy(
                tok.at[pl.ds(goff * S, GROUP_S * S)], ib, isem).wait()

            @pl.when(g + 1 < n_groups)
            def _():
                pltpu.async_copy(
                    tok.at[pl.ds((goff + GROUP_S) * S, GROUP_S * S)],
                    inext, isem_next)

            for t in range(4):
                pltpu.async_copy(tabs[t].at[ib.at[pl.ds(0, SUB_S * S)]],
                                 rbufs[t][0], gsems[t][0])

            @pl.loop(0, n_sub, step=2)
            def sub_loop(k0):
                for slot in range(2):
                    kk = k0 + slot
                    for t in range(4):
                        pltpu.make_async_copy(
                            tabs[t].at[ib.at[pl.ds(kk * SUB_S * S, SUB_S * S)]],
                            rbufs[t][slot], gsems[t][slot]).wait()

                    @pl.when(kk + 1 < n_sub)
                    def _():
                        for t in range(4):
                            pltpu.async_copy(
                                tabs[t].at[
                                    ib.at[pl.ds((kk + 1) * SUB_S * S, SUB_S * S)]],
                                rbufs[t][1 - slot], gsems[t][1 - slot])

                    for t in range(4):
                        compute_sub(rbufs[t][slot], ovs[t], kk * SUB_S)

            for t in range(4):
                pltpu.sync_copy(ovs[t], outs[t].at[pl.ds(goff, GROUP_S)])

    n_subq = n_qsent_w // SUB_S
    qbase = wid * n_qsent_w
    pltpu.sync_copy(qtok.at[pl.ds(qbase * Q, n_qsent_w * Q)],
                    i0.at[pl.ds(0, n_qsent_w * Q)])
    pltpu.async_copy(bw.at[i0.at[pl.ds(0, SUB_S * Q)]], rbufs[0][0],
                     gsems[0][0])

    @pl.loop(0, n_subq, step=2)
    def qsub_loop(k0):
        for slot in range(2):
            kk = k0 + slot
            pltpu.make_async_copy(
                bw.at[i0.at[pl.ds(kk * SUB_S * Q, SUB_S * Q)]],
                rbufs[0][slot], gsems[0][slot]).wait()

            @pl.when(kk + 1 < n_subq)
            def _():
                pltpu.async_copy(
                    bw.at[i0.at[pl.ds((kk + 1) * SUB_S * Q, SUB_S * Q)]],
                    rbufs[0][1 - slot], gsems[0][1 - slot])

            compute_sub(rbufs[0][slot], ovs[0], kk * SUB_S)

    pltpu.sync_copy(ovs[0].at[pl.ds(0, n_qsent_w)],
                    u0.at[pl.ds(qbase, n_qsent_w)])


def _sc_embed(tok, qtok, t0, t1, t2, t3, bw, pvec, qvec, rdv,
              n_sent, n_qsent, S, Q, D):
    n_sent_w = n_sent // NW
    n_qsent_w = n_qsent // NW
    mesh = plsc.VectorSubcoreMesh(core_axis_name="c", subcore_axis_name="s")
    out_type = ([jax.ShapeDtypeStruct((n_sent, D), jnp.float32)] * 4
                + [jax.ShapeDtypeStruct((n_qsent, D), jnp.float32)])
    scratch = (
        [pltpu.VMEM((GROUP_S * S,), jnp.int32)] * 2
        + [pltpu.VMEM((SUB_S * S, D), jnp.float32)] * 8
        + [pltpu.VMEM((GROUP_S, D), jnp.float32)] * 4
        + [pltpu.VMEM((S, 16), jnp.float32)] * 2
        + [pltpu.VMEM((D,), jnp.float32)]
        + [pltpu.SemaphoreType.DMA] * 10
    )
    body = functools.partial(_sc_embed_body, S, Q, n_sent_w, n_qsent_w)
    fn = pl.kernel(body, out_type=out_type, mesh=mesh, scratch_types=scratch,
                   compiler_params=pltpu.CompilerParams(
                       use_tc_tiling_on_sc=False))
    return fn(tok, qtok, t0, t1, t2, t3, bw, pvec, qvec, rdv)


def _hops_body(e0, e1, e2, e3, story_ref, u_ref, ta_ref, tc_ref, out_ref):
    u = u_ref[...]
    nonpad = jnp.any(story_ref[...] != 0, axis=2)
    es = [e0[...], e1[...], e2[...], e3[...]]
    ta = ta_ref[...]
    tc = tc_ref[...]
    for k in range(3):
        m = es[k] + ta[None]
        c = es[k + 1] + tc[None]
        scores = jnp.sum(m * u[:, None, :], axis=2)
        scores = jnp.where(nonpad, scores, NEG_INF)
        p = jax.nn.softmax(scores, axis=1)
        o = jnp.sum(p[:, :, None] * c, axis=1)
        u = u + o
    out_ref[...] = u


def _hops(e0, e1, e2, e3, story, u0, ta, tc, interpret=False):
    B, M, D = e0.shape
    S = story.shape[2]
    BB = 64
    grid = B // BB
    espec = pl.BlockSpec((BB, M, D), lambda i: (i, 0, 0))
    return pl.pallas_call(
        _hops_body,
        grid=(grid,),
        in_specs=[espec, espec, espec, espec,
                  pl.BlockSpec((BB, M, S), lambda i: (i, 0, 0)),
                  pl.BlockSpec((BB, D), lambda i: (i, 0)),
                  pl.BlockSpec((M, D), lambda i: (0, 0)),
                  pl.BlockSpec((M, D), lambda i: (0, 0))],
        out_specs=pl.BlockSpec((BB, D), lambda i: (i, 0)),
        out_shape=jax.ShapeDtypeStruct((B, D), jnp.float32),
        interpret=interpret,
    )(e0, e1, e2, e3, story, u0, ta, tc)


def _mm_body(u_ref, w_ref, b_ref, out_ref):
    out_ref[...] = lax.dot_general(
        u_ref[...], w_ref[...], (((1,), (1,)), ((), ())),
        preferred_element_type=jnp.float32) + b_ref[...]


def _classifier(u, W, b2d, interpret=False):
    B, D = u.shape
    V = W.shape[0]
    VB = 2048
    grid = pl.cdiv(V, VB)
    return pl.pallas_call(
        _mm_body,
        grid=(grid,),
        in_specs=[pl.BlockSpec((B, D), lambda v: (0, 0)),
                  pl.BlockSpec((VB, D), lambda v: (v, 0)),
                  pl.BlockSpec((1, VB), lambda v: (0, v))],
        out_specs=pl.BlockSpec((B, VB), lambda v: (0, v)),
        out_shape=jax.ShapeDtypeStruct((B, V), jnp.float32),
        interpret=interpret,
    )(u, W, b2d)


def kernel(story, question, A0, C0, C1, C2, Bw, TA, TC, W, b):
    B, M, S = story.shape
    Q = question.shape[1]
    V, D = A0.shape
    assert Q == S, "shared position-encoding factors assume Q == S"

    j = jnp.arange(1, S + 1, dtype=jnp.float32)
    p = 1.0 - j / S
    q = 1.0 - 2.0 * j / S
    kk = jnp.arange(1, D + 1, dtype=jnp.float32)
    r = -(kk / D)
    pvec = jnp.broadcast_to(p[:, None], (S, 16))
    qvec = jnp.broadcast_to(q[:, None], (S, 16))

    tok = story.reshape(B * M * S)
    qtok = question.reshape(B * Q)

    e0, e1, e2, e3, u0 = _sc_embed(tok, qtok, A0, C0, C1, C2, Bw,
                                   pvec, qvec, r, B * M, B, S, Q, D)

    u = _hops(e0.reshape(B, M, D), e1.reshape(B, M, D),
              e2.reshape(B, M, D), e3.reshape(B, M, D),
              story, u0, TA, TC)

    return _classifier(u, W, b.reshape(1, V))


# speedup vs baseline: 14.0161x; 1.2737x over previous
import functools

import jax
import jax.numpy as jnp
from jax import lax
from jax.experimental import pallas as pl
from jax.experimental.pallas import tpu as pltpu
from jax.experimental.pallas import tpu_sc as plsc

NEG_INF = -1000000000.0

NC = 2
NS = 16
NW = NC * NS

SUB_S = 4
GROUP_S = 80


def _sc_embed_body(S, Q, n_sent_w, n_qsent_w,
                   tok, qtok, t0, t1, t2, t3, bw, pvec_h, qvec_h, rd_h,
                   e0, e1, e2, e3, u0,
                   i0, i1, rA0, rA1, rA2, rA3, rB0, rB1, rB2, rB3,
                   o0, o1, o2, o3, pp, qq, rdm,
                   si0, si1, sA0, sA1, sA2, sA3, sB0, sB1, sB2, sB3):
    wid = lax.axis_index("s") * NC + lax.axis_index("c")

    pltpu.sync_copy(pvec_h, pp)
    pltpu.sync_copy(qvec_h, qq)
    pltpu.sync_copy(rd_h, rdm)
    rd = [rdm[pl.ds(c * 16, 16)] for c in range(4)]

    tabs = [t0, t1, t2, t3]
    outs = [e0, e1, e2, e3]
    ovs = [o0, o1, o2, o3]
    rbufs = [(rA0, rB0), (rA1, rB1), (rA2, rB2), (rA3, rB3)]
    gsems = [(sA0, sB0), (sA1, sB1), (sA2, sB2), (sA3, sB3)]

    zero = jnp.zeros((16,), jnp.float32)

    def compute_sub(buf, ov, row0):
        init = (tuple(zero for _ in range(SUB_S * 4)),
                tuple(zero for _ in range(SUB_S * 4)))

        def s_body(s, carry):
            accp, accq = list(carry[0]), list(carry[1])
            ps = pp[s, :]
            qs = qq[s, :]
            for j in range(SUB_S):
                for c in range(4):
                    row = buf[j * S + s, pl.ds(c * 16, 16)]
                    a = j * 4 + c
                    accp[a] = accp[a] + row * ps
                    accq[a] = accq[a] + row * qs
            return (tuple(accp), tuple(accq))

        accp, accq = lax.fori_loop(0, S, s_body, init)
        for j in range(SUB_S):
            for c in range(4):
                a = j * 4 + c
                ov[row0 + j, pl.ds(c * 16, 16)] = accp[a] + rd[c] * accq[a]

    n_groups = n_sent_w // GROUP_S
    n_sub = GROUP_S // SUB_S
    base_sent = wid * n_sent_w
    ibufs = (i0, i1)
    isems = (si0, si1)

    pltpu.async_copy(tok.at[pl.ds(base_sent * S, GROUP_S * S)], i0, si0)

    @pl.loop(0, n_groups, step=2)
    def group_loop(g0):
        for half in range(2):
            g = g0 + half
            ib, isem = ibufs[half], isems[half]
            inext, isem_next = ibufs[1 - half], isems[1 - half]
            goff = base_sent + g * GROUP_S
            pltpu.make_async_copy(
                tok.at[pl.ds(goff * S, GROUP_S * S)], ib, isem).wait()

            @pl.when(g + 1 < n_groups)
            def _():
                pltpu.async_copy(
                    tok.at[pl.ds((goff + GROUP_S) * S, GROUP_S * S)],
                    inext, isem_next)

            for t in range(4):
                pltpu.async_copy(tabs[t].at[ib.at[pl.ds(0, SUB_S * S)]],
                                 rbufs[t][0], gsems[t][0])

            @pl.loop(0, n_sub, step=2)
            def sub_loop(k0):
                for slot in range(2):
                    kk = k0 + slot
                    for t in range(4):
                        pltpu.make_async_copy(
                            tabs[t].at[ib.at[pl.ds(kk * SUB_S * S, SUB_S * S)]],
                            rbufs[t][slot], gsems[t][slot]).wait()

                    @pl.when(kk + 1 < n_sub)
                    def _():
                        for t in range(4):
                            pltpu.async_copy(
                                tabs[t].at[
                                    ib.at[pl.ds((kk + 1) * SUB_S * S, SUB_S * S)]],
                                rbufs[t][1 - slot], gsems[t][1 - slot])

                    for t in range(4):
                        compute_sub(rbufs[t][slot], ovs[t], kk * SUB_S)

            for t in range(4):
                pltpu.sync_copy(ovs[t], outs[t].at[pl.ds(goff, GROUP_S)])

    n_subq = n_qsent_w // SUB_S
    qbase = wid * n_qsent_w
    pltpu.sync_copy(qtok.at[pl.ds(qbase * Q, n_qsent_w * Q)],
                    i0.at[pl.ds(0, n_qsent_w * Q)])
    pltpu.async_copy(bw.at[i0.at[pl.ds(0, SUB_S * Q)]], rbufs[0][0],
                     gsems[0][0])

    @pl.loop(0, n_subq, step=2)
    def qsub_loop(k0):
        for slot in range(2):
            kk = k0 + slot
            pltpu.make_async_copy(
                bw.at[i0.at[pl.ds(kk * SUB_S * Q, SUB_S * Q)]],
                rbufs[0][slot], gsems[0][slot]).wait()

            @pl.when(kk + 1 < n_subq)
            def _():
                pltpu.async_copy(
                    bw.at[i0.at[pl.ds((kk + 1) * SUB_S * Q, SUB_S * Q)]],
                    rbufs[0][1 - slot], gsems[0][1 - slot])

            compute_sub(rbufs[0][slot], ovs[0], kk * SUB_S)

    pltpu.sync_copy(ovs[0].at[pl.ds(0, n_qsent_w)],
                    u0.at[pl.ds(qbase, n_qsent_w)])


def _sc_embed(tok, qtok, t0, t1, t2, t3, bw, pvec, qvec, rdv,
              n_sent, n_qsent, S, Q, D):
    n_sent_w = n_sent // NW
    n_qsent_w = n_qsent // NW
    mesh = plsc.VectorSubcoreMesh(core_axis_name="c", subcore_axis_name="s")
    out_type = ([jax.ShapeDtypeStruct((n_sent, D), jnp.float32)] * 4
                + [jax.ShapeDtypeStruct((n_qsent, D), jnp.float32)])
    scratch = (
        [pltpu.VMEM((GROUP_S * S,), jnp.int32)] * 2
        + [pltpu.VMEM((SUB_S * S, D), jnp.float32)] * 8
        + [pltpu.VMEM((GROUP_S, D), jnp.float32)] * 4
        + [pltpu.VMEM((S, 16), jnp.float32)] * 2
        + [pltpu.VMEM((D,), jnp.float32)]
        + [pltpu.SemaphoreType.DMA] * 10
    )
    body = functools.partial(_sc_embed_body, S, Q, n_sent_w, n_qsent_w)
    fn = pl.kernel(body, out_type=out_type, mesh=mesh, scratch_types=scratch,
                   compiler_params=pltpu.CompilerParams(
                       use_tc_tiling_on_sc=False))
    return fn(tok, qtok, t0, t1, t2, t3, bw, pvec, qvec, rdv)


def _hops_body(e0, e1, e2, e3, story_ref, u_ref, ta_ref, tc_ref, out_ref):
    u = u_ref[...]
    nonpad = jnp.any(story_ref[...] != 0, axis=2)
    es = [e0[...], e1[...], e2[...], e3[...]]
    ta = ta_ref[...]
    tc = tc_ref[...]
    for k in range(3):
        m = es[k] + ta[None]
        c = es[k + 1] + tc[None]
        scores = jnp.sum(m * u[:, None, :], axis=2)
        scores = jnp.where(nonpad, scores, NEG_INF)
        p = jax.nn.softmax(scores, axis=1)
        o = jnp.sum(p[:, :, None] * c, axis=1)
        u = u + o
    out_ref[...] = u


def _hops(e0, e1, e2, e3, story, u0, ta, tc, interpret=False):
    B, M, D = e0.shape
    S = story.shape[2]
    BB = 64
    grid = B // BB
    espec = pl.BlockSpec((BB, M, D), lambda i: (i, 0, 0))
    return pl.pallas_call(
        _hops_body,
        grid=(grid,),
        in_specs=[espec, espec, espec, espec,
                  pl.BlockSpec((BB, M, S), lambda i: (i, 0, 0)),
                  pl.BlockSpec((BB, D), lambda i: (i, 0)),
                  pl.BlockSpec((M, D), lambda i: (0, 0)),
                  pl.BlockSpec((M, D), lambda i: (0, 0))],
        out_specs=pl.BlockSpec((BB, D), lambda i: (i, 0)),
        out_shape=jax.ShapeDtypeStruct((B, D), jnp.float32),
        interpret=interpret,
    )(e0, e1, e2, e3, story, u0, ta, tc)


def _mm_body(u_ref, w_ref, b_ref, out_ref):
    out_ref[...] = lax.dot_general(
        w_ref[...], u_ref[...], (((0,), (1,)), ((), ())),
        preferred_element_type=jnp.float32) + b_ref[...]


def _classifier(u, W, b2d, interpret=False):
    B, D = u.shape
    V = W.shape[0]
    VB = 2048
    grid = pl.cdiv(V, VB)
    wt = jnp.transpose(W)
    lt = pl.pallas_call(
        _mm_body,
        grid=(grid,),
        in_specs=[pl.BlockSpec((B, D), lambda v: (0, 0)),
                  pl.BlockSpec((D, VB), lambda v: (0, v)),
                  pl.BlockSpec((VB, 1), lambda v: (v, 0))],
        out_specs=pl.BlockSpec((VB, B), lambda v: (v, 0)),
        out_shape=jax.ShapeDtypeStruct((V, B), jnp.float32),
        interpret=interpret,
    )(u, wt, b2d)
    return jnp.transpose(lt)


def kernel(story, question, A0, C0, C1, C2, Bw, TA, TC, W, b):
    B, M, S = story.shape
    Q = question.shape[1]
    V, D = A0.shape
    assert Q == S, "shared position-encoding factors assume Q == S"

    j = jnp.arange(1, S + 1, dtype=jnp.float32)
    p = 1.0 - j / S
    q = 1.0 - 2.0 * j / S
    kk = jnp.arange(1, D + 1, dtype=jnp.float32)
    r = -(kk / D)
    pvec = jnp.broadcast_to(p[:, None], (S, 16))
    qvec = jnp.broadcast_to(q[:, None], (S, 16))

    tok = story.reshape(B * M * S)
    qtok = question.reshape(B * Q)

    e0, e1, e2, e3, u0 = _sc_embed(tok, qtok, A0, C0, C1, C2, Bw,
                                   pvec, qvec, r, B * M, B, S, Q, D)

    u = _hops(e0.reshape(B, M, D), e1.reshape(B, M, D),
              e2.reshape(B, M, D), e3.reshape(B, M, D),
              story, u0, TA, TC)

    return _classifier(u, W, b.reshape(V, 1))
